# baseline (speedup 1.0000x reference)
import functools

import jax
import jax.numpy as jnp
from jax.experimental import pallas as pl
from jax.experimental.pallas import tpu as pltpu

_UNROLL = 8


def _lstm_layer_kernel(xf_ref, xb_ref, wx_ref, wh_ref, b_ref,
                       outf_ref, outb_ref, xgf_s, xgb_s, c_s, h_s,
                       *, tb: int, bsz: int, hdim: int, out_dtype):
    t_blk = pl.program_id(0)

    xgf_s[...] = (
        jnp.dot(xf_ref[0].reshape(tb * bsz, xf_ref.shape[-1]), wx_ref[0],
                preferred_element_type=jnp.float32) + b_ref[0]
    ).astype(jnp.bfloat16)
    xgb_s[...] = (
        jnp.dot(xb_ref[0].reshape(tb * bsz, xb_ref.shape[-1]), wx_ref[1],
                preferred_element_type=jnp.float32) + b_ref[1]
    ).astype(jnp.bfloat16)

    @pl.when(t_blk == 0)
    def _():
        c_s[...] = jnp.zeros_like(c_s)
        h_s[...] = jnp.zeros_like(h_s)

    def step(xg_s, out_ref, row, c, h):
        hb = h.astype(jnp.bfloat16)
        zs = []
        for g in range(4):
            zs.append(
                jnp.dot(hb, wh_ref[0, :, g * hdim:(g + 1) * hdim],
                        preferred_element_type=jnp.float32)
                + xg_s[pl.ds(row, bsz), g * hdim:(g + 1) * hdim]
            )
        zi, zf, zg, zo = zs
        c = jax.nn.sigmoid(zf) * c + jax.nn.sigmoid(zi) * jnp.tanh(zg)
        h = jax.nn.sigmoid(zo) * jnp.tanh(c)
        out_ref[pl.ds(row, bsz)] = h.astype(out_dtype)
        return c, h

    def stepb(xg_s, out_ref, row, c, h):
        hb = h.astype(jnp.bfloat16)
        zs = []
        for g in range(4):
            zs.append(
                jnp.dot(hb, wh_ref[1, :, g * hdim:(g + 1) * hdim],
                        preferred_element_type=jnp.float32)
                + xg_s[pl.ds(row, bsz), g * hdim:(g + 1) * hdim]
            )
        zi, zf, zg, zo = zs
        c = jax.nn.sigmoid(zf) * c + jax.nn.sigmoid(zi) * jnp.tanh(zg)
        h = jax.nn.sigmoid(zo) * jnp.tanh(c)
        out_ref[pl.ds(row, bsz)] = h.astype(out_dtype)
        return c, h

    def body(i, carry):
        cf, hf, cb, hb = carry
        for u in range(_UNROLL):
            t = i * _UNROLL + u
            cf, hf = step(xgf_s, outf_ref, t * bsz, cf, hf)
            cb, hb = stepb(xgb_s, outb_ref, (tb - 1 - t) * bsz, cb, hb)
        return cf, hf, cb, hb

    out = jax.lax.fori_loop(0, tb // _UNROLL, body,
                            (c_s[0], h_s[0], c_s[1], h_s[1]))
    c_s[0], h_s[0], c_s[1], h_s[1] = out


def _bidir_lstm_layer(x2, wx, wh, b, tb: int, out_dtype):
    _, t, bsz, d_in = x2.shape
    hdim = wh.shape[1]
    n_blk = t // tb
    kern = functools.partial(_lstm_layer_kernel, tb=tb, bsz=bsz, hdim=hdim,
                             out_dtype=out_dtype)
    fwd, bwz = pl.pallas_call(
        kern,
        grid=(n_blk,),
        in_specs=[
            pl.BlockSpec((1, tb, bsz, d_in), lambda i: (0, i, 0, 0)),
            pl.BlockSpec((1, tb, bsz, d_in), lambda i: (1, n_blk - 1 - i, 0, 0)),
            pl.BlockSpec((2, d_in, 4 * hdim), lambda i: (0, 0, 0)),
            pl.BlockSpec((2, hdim, 4 * hdim), lambda i: (0, 0, 0)),
            pl.BlockSpec((2, 1, 4 * hdim), lambda i: (0, 0, 0)),
        ],
        out_specs=[
            pl.BlockSpec((tb * bsz, hdim), lambda i: (i, 0)),
            pl.BlockSpec((tb * bsz, hdim), lambda i: (n_blk - 1 - i, 0)),
        ],
        out_shape=[
            jax.ShapeDtypeStruct((t * bsz, hdim), out_dtype),
            jax.ShapeDtypeStruct((t * bsz, hdim), out_dtype),
        ],
        scratch_shapes=[
            pltpu.VMEM((tb * bsz, 4 * hdim), jnp.bfloat16),
            pltpu.VMEM((tb * bsz, 4 * hdim), jnp.bfloat16),
            pltpu.VMEM((2, bsz, hdim), jnp.float32),
            pltpu.VMEM((2, bsz, hdim), jnp.float32),
        ],
        compiler_params=pltpu.CompilerParams(
            dimension_semantics=("arbitrary",),
            vmem_limit_bytes=100 * 1024 * 1024,
        ),
    )(x2, x2, wx, wh, b[:, None, :])
    return fwd.reshape(t, bsz, hdim), bwz.reshape(t, bsz, hdim)


def _roll_kernel(s_ref, v_ref, o_ref, diff_s, p_s, *, t: int):
    b = pl.program_id(0)

    @pl.when(b == 0)
    def _():
        ti = jax.lax.broadcasted_iota(jnp.int32, (t, t), 0)
        ui = jax.lax.broadcasted_iota(jnp.int32, (t, t), 1)
        diff_s[...] = jax.lax.rem(ui - ti + t, t).astype(jnp.float32)

    s = s_ref[b].astype(jnp.float32)
    p_s[...] = jnp.where(diff_s[...] == s, 1.0, 0.0)
    o_ref[...] = jnp.dot(p_s[...], v_ref[...].astype(jnp.float32),
                         preferred_element_type=jnp.float32
                         ).astype(jnp.bfloat16)


def _roll_tm(v, starts):
    t, bsz, dim = v.shape
    vr = v.reshape(t, bsz * dim)
    out = pl.pallas_call(
        functools.partial(_roll_kernel, t=t),
        grid_spec=pltpu.PrefetchScalarGridSpec(
            num_scalar_prefetch=1,
            grid=(bsz,),
            in_specs=[pl.BlockSpec((t, dim), lambda b, s: (0, b))],
            out_specs=pl.BlockSpec((t, dim), lambda b, s: (0, b)),
            scratch_shapes=[
                pltpu.VMEM((t, t), jnp.float32),
                pltpu.VMEM((t, t), jnp.float32),
            ],
        ),
        out_shape=jax.ShapeDtypeStruct((t, bsz * dim), jnp.bfloat16),
        compiler_params=pltpu.CompilerParams(
            dimension_semantics=("arbitrary",),
            vmem_limit_bytes=100 * 1024 * 1024,
        ),
    )(starts, vr)
    return out.reshape(t, bsz, dim)


def kernel(inputs, input_paddings, Wx, Wh, b):
    t = inputs.shape[1]
    tb = 32 if t % 32 == 0 else t
    lengths = jnp.sum(1.0 - input_paddings, axis=-1).astype(jnp.int32)
    s_len = jax.lax.rem(lengths, t)
    s_neg = jax.lax.rem(t - lengths, t)

    x_tm = inputs.transpose(1, 0, 2).astype(jnp.bfloat16)
    z0 = _roll_tm(x_tm, s_len)
    x2 = jnp.stack([x_tm, z0])

    wx2 = Wx.astype(jnp.bfloat16)
    wh2 = Wh.astype(jnp.bfloat16)

    n_layers = Wx.shape[0]
    for l in range(n_layers):
        is_last = l + 1 == n_layers
        fwd, bwz = _bidir_lstm_layer(x2, wx2[l], wh2[l], b[l], tb,
                                     jnp.bfloat16)
        if not is_last:
            x2 = jnp.stack([
                jnp.concatenate([fwd, _roll_tm(bwz, s_neg)], axis=-1),
                jnp.concatenate([_roll_tm(fwd, s_len), bwz], axis=-1),
            ])
    out = jnp.concatenate([fwd, _roll_tm(bwz, s_neg)], axis=-1)
    return out.astype(jnp.float32).transpose(1, 0, 2)


# speedup vs baseline: 1.0278x; 1.0278x over previous
import functools

import jax
import jax.numpy as jnp
from jax.experimental import pallas as pl
from jax.experimental.pallas import tpu as pltpu

_UNROLL = 8


def _lstm_layer_kernel(xf_ref, xb_ref, wx_ref, wh_ref, b_ref,
                       outf_ref, outb_ref, xgf_s, xgb_s, c_s, h_s,
                       *, tb: int, bsz: int, hdim: int, out_dtype):
    t_blk = pl.program_id(0)

    xgf_s[...] = (
        jnp.dot(xf_ref[0].reshape(tb * bsz, xf_ref.shape[-1]), wx_ref[0],
                preferred_element_type=jnp.float32) + b_ref[0]
    ).astype(jnp.bfloat16)
    xgb_s[...] = (
        jnp.dot(xb_ref[0].reshape(tb * bsz, xb_ref.shape[-1]), wx_ref[1],
                preferred_element_type=jnp.float32) + b_ref[1]
    ).astype(jnp.bfloat16)

    @pl.when(t_blk == 0)
    def _():
        c_s[...] = jnp.zeros_like(c_s)
        h_s[...] = jnp.zeros_like(h_s)

    def step(xg_s, out_ref, row, c, h):
        hb = h.astype(jnp.bfloat16)
        zs = []
        for g in range(4):
            zs.append(
                jnp.dot(hb, wh_ref[0, :, g * hdim:(g + 1) * hdim],
                        preferred_element_type=jnp.float32)
                + xg_s[pl.ds(row, bsz), g * hdim:(g + 1) * hdim]
            )
        zi, zf, zg, zo = zs
        c = jax.nn.sigmoid(zf) * c + jax.nn.sigmoid(zi) * jnp.tanh(zg)
        h = jax.nn.sigmoid(zo) * jnp.tanh(c)
        out_ref[pl.ds(row, bsz)] = h.astype(out_dtype)
        return c, h

    def stepb(xg_s, out_ref, row, c, h):
        hb = h.astype(jnp.bfloat16)
        zs = []
        for g in range(4):
            zs.append(
                jnp.dot(hb, wh_ref[1, :, g * hdim:(g + 1) * hdim],
                        preferred_element_type=jnp.float32)
                + xg_s[pl.ds(row, bsz), g * hdim:(g + 1) * hdim]
            )
        zi, zf, zg, zo = zs
        c = jax.nn.sigmoid(zf) * c + jax.nn.sigmoid(zi) * jnp.tanh(zg)
        h = jax.nn.sigmoid(zo) * jnp.tanh(c)
        out_ref[pl.ds(row, bsz)] = h.astype(out_dtype)
        return c, h

    def body(i, carry):
        cf, hf, cb, hb = carry
        for u in range(_UNROLL):
            t = i * _UNROLL + u
            cf, hf = step(xgf_s, outf_ref, t * bsz, cf, hf)
            cb, hb = stepb(xgb_s, outb_ref, (tb - 1 - t) * bsz, cb, hb)
        return cf, hf, cb, hb

    out = jax.lax.fori_loop(0, tb // _UNROLL, body,
                            (c_s[0], h_s[0], c_s[1], h_s[1]))
    c_s[0], h_s[0], c_s[1], h_s[1] = out


def _lstm_layer_kernel4(fl_ref, fr_ref, bl_ref, br_ref, wx_ref, wh_ref, b_ref,
                        outf_ref, outb_ref, xgf_s, xgb_s, c_s, h_s,
                        *, tb: int, bsz: int, hdim: int, out_dtype):
    t_blk = pl.program_id(0)
    k = fl_ref.shape[-1]

    xgf_s[...] = (
        jnp.dot(fl_ref[...], wx_ref[0, :k, :],
                preferred_element_type=jnp.float32)
        + jnp.dot(fr_ref[...], wx_ref[0, k:, :],
                  preferred_element_type=jnp.float32)
        + b_ref[0]
    ).astype(jnp.bfloat16)
    xgb_s[...] = (
        jnp.dot(bl_ref[...], wx_ref[1, :k, :],
                preferred_element_type=jnp.float32)
        + jnp.dot(br_ref[...], wx_ref[1, k:, :],
                  preferred_element_type=jnp.float32)
        + b_ref[1]
    ).astype(jnp.bfloat16)

    @pl.when(t_blk == 0)
    def _():
        c_s[...] = jnp.zeros_like(c_s)
        h_s[...] = jnp.zeros_like(h_s)

    def step(wd, xg_s, out_ref, row, c, h):
        hb = h.astype(jnp.bfloat16)
        zs = []
        for g in range(4):
            zs.append(
                jnp.dot(hb, wh_ref[wd, :, g * hdim:(g + 1) * hdim],
                        preferred_element_type=jnp.float32)
                + xg_s[pl.ds(row, bsz), g * hdim:(g + 1) * hdim]
            )
        zi, zf, zg, zo = zs
        c = jax.nn.sigmoid(zf) * c + jax.nn.sigmoid(zi) * jnp.tanh(zg)
        h = jax.nn.sigmoid(zo) * jnp.tanh(c)
        out_ref[pl.ds(row, bsz)] = h.astype(out_dtype)
        return c, h

    def body(i, carry):
        cf, hf, cb, hb = carry
        for u in range(_UNROLL):
            t = i * _UNROLL + u
            cf, hf = step(0, xgf_s, outf_ref, t * bsz, cf, hf)
            cb, hb = step(1, xgb_s, outb_ref, (tb - 1 - t) * bsz, cb, hb)
        return cf, hf, cb, hb

    out = jax.lax.fori_loop(0, tb // _UNROLL, body,
                            (c_s[0], h_s[0], c_s[1], h_s[1]))
    c_s[0], h_s[0], c_s[1], h_s[1] = out


def _bidir_lstm_layer4(fl, fr, bl, br, wx, wh, b, t: int, bsz: int,
                       tb: int, out_dtype):
    hdim = wh.shape[1]
    n_blk = t // tb
    kern = functools.partial(_lstm_layer_kernel4, tb=tb, bsz=bsz, hdim=hdim,
                             out_dtype=out_dtype)
    fmap = lambda i: (i, 0)
    bmap = lambda i: (n_blk - 1 - i, 0)
    blk = (tb * bsz, hdim)
    fwd, bwz = pl.pallas_call(
        kern,
        grid=(n_blk,),
        in_specs=[
            pl.BlockSpec(blk, fmap),
            pl.BlockSpec(blk, fmap),
            pl.BlockSpec(blk, bmap),
            pl.BlockSpec(blk, bmap),
            pl.BlockSpec((2, 2 * hdim, 4 * hdim), lambda i: (0, 0, 0)),
            pl.BlockSpec((2, hdim, 4 * hdim), lambda i: (0, 0, 0)),
            pl.BlockSpec((2, 1, 4 * hdim), lambda i: (0, 0, 0)),
        ],
        out_specs=[
            pl.BlockSpec(blk, fmap),
            pl.BlockSpec(blk, bmap),
        ],
        out_shape=[
            jax.ShapeDtypeStruct((t * bsz, hdim), out_dtype),
            jax.ShapeDtypeStruct((t * bsz, hdim), out_dtype),
        ],
        scratch_shapes=[
            pltpu.VMEM((tb * bsz, 4 * hdim), jnp.bfloat16),
            pltpu.VMEM((tb * bsz, 4 * hdim), jnp.bfloat16),
            pltpu.VMEM((2, bsz, hdim), jnp.float32),
            pltpu.VMEM((2, bsz, hdim), jnp.float32),
        ],
        compiler_params=pltpu.CompilerParams(
            dimension_semantics=("arbitrary",),
            vmem_limit_bytes=100 * 1024 * 1024,
        ),
    )(fl, fr, bl, br, wx, wh, b[:, None, :])
    return fwd, bwz


def _bidir_lstm_layer(x2, wx, wh, b, tb: int, out_dtype):
    _, t, bsz, d_in = x2.shape
    hdim = wh.shape[1]
    n_blk = t // tb
    kern = functools.partial(_lstm_layer_kernel, tb=tb, bsz=bsz, hdim=hdim,
                             out_dtype=out_dtype)
    fwd, bwz = pl.pallas_call(
        kern,
        grid=(n_blk,),
        in_specs=[
            pl.BlockSpec((1, tb, bsz, d_in), lambda i: (0, i, 0, 0)),
            pl.BlockSpec((1, tb, bsz, d_in), lambda i: (1, n_blk - 1 - i, 0, 0)),
            pl.BlockSpec((2, d_in, 4 * hdim), lambda i: (0, 0, 0)),
            pl.BlockSpec((2, hdim, 4 * hdim), lambda i: (0, 0, 0)),
            pl.BlockSpec((2, 1, 4 * hdim), lambda i: (0, 0, 0)),
        ],
        out_specs=[
            pl.BlockSpec((tb * bsz, hdim), lambda i: (i, 0)),
            pl.BlockSpec((tb * bsz, hdim), lambda i: (n_blk - 1 - i, 0)),
        ],
        out_shape=[
            jax.ShapeDtypeStruct((t * bsz, hdim), out_dtype),
            jax.ShapeDtypeStruct((t * bsz, hdim), out_dtype),
        ],
        scratch_shapes=[
            pltpu.VMEM((tb * bsz, 4 * hdim), jnp.bfloat16),
            pltpu.VMEM((tb * bsz, 4 * hdim), jnp.bfloat16),
            pltpu.VMEM((2, bsz, hdim), jnp.float32),
            pltpu.VMEM((2, bsz, hdim), jnp.float32),
        ],
        compiler_params=pltpu.CompilerParams(
            dimension_semantics=("arbitrary",),
            vmem_limit_bytes=100 * 1024 * 1024,
        ),
    )(x2, x2, wx, wh, b[:, None, :])
    return fwd.reshape(t, bsz, hdim), bwz.reshape(t, bsz, hdim)


def _roll_kernel(s_ref, v_ref, o_ref, diff_s, p_s, *, t: int):
    b = pl.program_id(0)

    @pl.when(b == 0)
    def _():
        ti = jax.lax.broadcasted_iota(jnp.int32, (t, t), 0)
        ui = jax.lax.broadcasted_iota(jnp.int32, (t, t), 1)
        diff_s[...] = jax.lax.rem(ui - ti + t, t).astype(jnp.float32)

    s = s_ref[b].astype(jnp.float32)
    p_s[...] = jnp.where(diff_s[...] == s, 1.0, 0.0)
    o_ref[...] = jnp.dot(p_s[...], v_ref[...].astype(jnp.float32),
                         preferred_element_type=jnp.float32
                         ).astype(jnp.bfloat16)


def _roll_tm(v, starts):
    t, bsz, dim = v.shape
    vr = v.reshape(t, bsz * dim)
    out = pl.pallas_call(
        functools.partial(_roll_kernel, t=t),
        grid_spec=pltpu.PrefetchScalarGridSpec(
            num_scalar_prefetch=1,
            grid=(bsz,),
            in_specs=[pl.BlockSpec((t, dim), lambda b, s: (0, b))],
            out_specs=pl.BlockSpec((t, dim), lambda b, s: (0, b)),
            scratch_shapes=[
                pltpu.VMEM((t, t), jnp.float32),
                pltpu.VMEM((t, t), jnp.float32),
            ],
        ),
        out_shape=jax.ShapeDtypeStruct((t, bsz * dim), jnp.bfloat16),
        compiler_params=pltpu.CompilerParams(
            dimension_semantics=("arbitrary",),
            vmem_limit_bytes=100 * 1024 * 1024,
        ),
    )(starts, vr)
    return out.reshape(t, bsz, dim)


def kernel(inputs, input_paddings, Wx, Wh, b):
    t = inputs.shape[1]
    tb = 32 if t % 32 == 0 else t
    lengths = jnp.sum(1.0 - input_paddings, axis=-1).astype(jnp.int32)
    s_len = jax.lax.rem(lengths, t)
    s_neg = jax.lax.rem(t - lengths, t)

    x_tm = inputs.transpose(1, 0, 2).astype(jnp.bfloat16)
    z0 = _roll_tm(x_tm, s_len)
    x2 = jnp.stack([x_tm, z0])

    wx2 = Wx.astype(jnp.bfloat16)
    wh2 = Wh.astype(jnp.bfloat16)
    bsz = inputs.shape[0]

    n_layers = Wx.shape[0]
    fwd, bwz = _bidir_lstm_layer(x2, wx2[0], wh2[0], b[0], tb, jnp.bfloat16)
    for l in range(1, n_layers):
        roll_bwz = _roll_tm(bwz, s_neg)
        roll_fwd = _roll_tm(fwd, s_len)
        hdim = bwz.shape[-1]
        fwd, bwz = _bidir_lstm_layer4(
            fwd.reshape(t * bsz, hdim), roll_bwz.reshape(t * bsz, hdim),
            roll_fwd.reshape(t * bsz, hdim), bwz.reshape(t * bsz, hdim),
            wx2[l], wh2[l], b[l], t, bsz, tb, jnp.bfloat16)
        fwd = fwd.reshape(t, bsz, hdim)
        bwz = bwz.reshape(t, bsz, hdim)
    out = jnp.concatenate([fwd, _roll_tm(bwz, s_neg)], axis=-1)
    return out.astype(jnp.float32).transpose(1, 0, 2)
